# baseline (speedup 1.0000x reference)
import functools

import jax
import jax.numpy as jnp
from jax import lax
from jax.experimental import pallas as pl
from jax.experimental.pallas import tpu as pltpu
from jax.experimental.pallas import tpu_sc as plsc

N_EMB = 1000000
D_EMB = 32
BATCH = 16384

L = 16
NC = 2
NS = 16
NW = NC * NS
B_PER_W = BATCH // NW
CHUNK = 128
N_CHUNKS = B_PER_W // CHUNK
GROUPS = B_PER_W // L


@functools.partial(
    pl.kernel,
    mesh=plsc.VectorSubcoreMesh(core_axis_name="c", subcore_axis_name="s"),
    out_type=jax.ShapeDtypeStruct((BATCH,), jnp.float32),
    scratch_types=[
        pltpu.VMEM((B_PER_W,), jnp.int32),
        pltpu.VMEM((N_CHUNKS, CHUNK), jnp.int32),
        pltpu.VMEM((B_PER_W, 1, D_EMB), jnp.float32),
        pltpu.VMEM(((D_EMB + 1) * L,), jnp.float32),
        pltpu.VMEM((B_PER_W,), jnp.float32),
        pltpu.SemaphoreType.DMA,
    ],
    compiler_params=pltpu.CompilerParams(needs_layout_passes=False),
)
def _sc_gather_dot(idx_hbm, table_hbm, wb_hbm, out_hbm,
                   idx_v, idx2_v, rows3, wb_v, out_v, sem):
    wid = lax.axis_index("s") * NC + lax.axis_index("c")
    base = wid * B_PER_W

    pltpu.sync_copy(idx_hbm.at[pl.ds(base, B_PER_W)], idx_v)
    for j in range(N_CHUNKS):
        pltpu.sync_copy(
            idx_hbm.at[pl.ds(base + j * CHUNK, CHUNK)], idx2_v.at[j])
    pltpu.sync_copy(wb_hbm, wb_v)

    copies = []
    for j in range(N_CHUNKS):
        copies.append(pltpu.async_copy(
            table_hbm.at[idx2_v.at[j]],
            rows3.at[pl.ds(j * CHUNK, CHUNK)],
            sem,
        ))
    for c in copies:
        c.wait()
    rows_v = rows3.reshape(B_PER_W, D_EMB)

    ws = [wb_v[pl.ds(d * L, L)] for d in range(D_EMB)]
    bias = wb_v[pl.ds(D_EMB * L, L)]
    lane = lax.iota(jnp.int32, L)

    def body(g, carry):
        row0 = g * L
        rid = lane + row0
        acc = bias
        for d in range(D_EMB):
            col = plsc.load_gather(
                rows_v, [rid, jnp.full((L,), d, dtype=jnp.int32)])
            acc = acc + col * ws[d]
        out_v[pl.ds(row0, L)] = acc
        return carry

    lax.fori_loop(0, GROUPS, body, 0)

    pltpu.sync_copy(out_v, out_hbm.at[pl.ds(base, B_PER_W)])


def kernel(idx, table, W, b):
    table_lin = table.reshape(N_EMB, 1, D_EMB)
    wb = jnp.concatenate(
        [
            jnp.broadcast_to(W.reshape(D_EMB, 1), (D_EMB, L)),
            jnp.broadcast_to(b.reshape(1, 1), (1, L)),
        ],
        axis=0,
    ).reshape((D_EMB + 1) * L)
    y = _sc_gather_dot(idx.astype(jnp.int32), table_lin, wb)
    return y.reshape(BATCH, 1)
